# baseline (speedup 1.0000x reference)
import functools
import math

import jax
import jax.numpy as jnp
from jax import lax
from jax.experimental import pallas as pl
from jax.experimental.pallas import tpu as pltpu
from jax.experimental.pallas import tpu_sc as plsc

VOCAB = 250027
D = 1024
B = 32
S = 1024
NC = 2
NS = 16
NW = NC * NS
SCALE = math.sqrt(float(D))
EPS = 1e-5
K = 4
SEG_S = S // K
CH = SEG_S // NW
SB = 64
SEG_BLOCKS = SEG_S // SB


def _sc_gather_body(tok_hbm, weight_hbm, out_hbm,
                    tok_v, buf0, buf1, gs0, gs1, ss0, ss1):
    wid = lax.axis_index("s") * NC + lax.axis_index("c")
    row0 = wid * CH * B

    pltpu.sync_copy(tok_hbm.at[pl.ds(wid * CH, CH)], tok_v)

    bufs = (buf0, buf1)
    gsems = (gs0, gs1)
    ssems = (ss0, ss1)

    pltpu.async_copy(weight_hbm.at[tok_v.at[0]], buf0, gs0)
    pltpu.async_copy(weight_hbm.at[tok_v.at[1]], buf1, gs1)

    def pair_body(i, _):
        for b in range(2):
            c = 2 * i + b
            buf, gs, ss = bufs[b], gsems[b], ssems[b]
            pltpu.make_async_copy(weight_hbm.at[tok_v.at[c]], buf, gs).wait()
            dst = out_hbm.at[pl.ds(row0 + c * B, B)]
            pltpu.async_copy(buf, dst, ss)

            @pl.when(c + 2 < CH)
            def _():
                pltpu.make_async_copy(buf, dst, ss).wait()
                pltpu.async_copy(weight_hbm.at[tok_v.at[c + 2]], buf, gs)

        return 0

    lax.fori_loop(0, CH // 2, pair_body, 0)

    for b in range(2):
        c = CH - 2 + b
        pltpu.make_async_copy(bufs[b], out_hbm.at[pl.ds(row0 + c * B, B)],
                              ssems[b]).wait()


def _sc_gather_seg(tokens_seg, weight):
    mesh = plsc.VectorSubcoreMesh(core_axis_name="c", subcore_axis_name="s")
    return pl.kernel(
        _sc_gather_body,
        mesh=mesh,
        out_type=jax.ShapeDtypeStruct((SEG_S * B, D), jnp.float32),
        scratch_types=[
            pltpu.VMEM((CH, B), jnp.int32),
            pltpu.VMEM((B, D), jnp.float32),
            pltpu.VMEM((B, D), jnp.float32),
            pltpu.SemaphoreType.DMA,
            pltpu.SemaphoreType.DMA,
            pltpu.SemaphoreType.DMA,
            pltpu.SemaphoreType.DMA,
        ],
    )(tokens_seg, weight)


def _tc_ln_body(*refs):
    pos_ref, gam_ref, bet_ref, g_ref, o_ref = refs[-5:]
    x = g_ref[...].reshape(SB, B, D) * SCALE + pos_ref[...][:, None, :]
    mean = jnp.mean(x, axis=-1, keepdims=True)
    xc = x - mean
    var = jnp.mean(xc * xc, axis=-1, keepdims=True)
    y = xc * lax.rsqrt(var + EPS) * gam_ref[...][None, :, :] + bet_ref[...]
    o_ref[...] = y.reshape(SB * B, D)


def _tc_ln_seg(k, carry, gath_seg, pos_weight, gamma2d, beta2d):
    specs = [
        pl.BlockSpec((SB, D), lambda i: (k * SEG_BLOCKS + i, 0)),
        pl.BlockSpec((1, D), lambda i: (0, 0)),
        pl.BlockSpec((1, D), lambda i: (0, 0)),
        pl.BlockSpec((SB * B, D), lambda i: (i, 0)),
    ]
    args = (pos_weight, gamma2d, beta2d, gath_seg)
    aliases = {}
    if carry is not None:
        specs = [pl.BlockSpec(memory_space=pl.ANY)] + specs
        args = (carry,) + args
        aliases = {0: 0}
    return pl.pallas_call(
        _tc_ln_body,
        grid=(SEG_BLOCKS,),
        in_specs=specs,
        out_specs=pl.BlockSpec((SB * B, D),
                               lambda i: (k * SEG_BLOCKS + i, 0)),
        out_shape=jax.ShapeDtypeStruct((S * B, D), jnp.float32),
        input_output_aliases=aliases,
        compiler_params=pltpu.CompilerParams(
            dimension_semantics=("arbitrary",)),
    )(*args)


@jax.jit
def _shard_head_tail(tokens, weight, pos_weight, ln_gamma, ln_beta):
    tokens_t = jnp.transpose(tokens)
    gamma2d = ln_gamma.reshape(1, D)
    beta2d = ln_beta.reshape(1, D)
    gath = [_sc_gather_seg(
        lax.slice_in_dim(tokens_t, k * SEG_S, (k + 1) * SEG_S), weight)
        for k in range(K)]
    out = None
    for k in range(K):
        out = _tc_ln_seg(k, out, gath[k], pos_weight, gamma2d, beta2d)
    return out.reshape(S, B, D)


def kernel(tokens, weight, pos_weight, ln_gamma, ln_beta):
    return _shard_head_tail(tokens, weight, pos_weight, ln_gamma, ln_beta)
